# baseline (speedup 1.0000x reference)
import functools

import jax
import jax.numpy as jnp
from jax import lax
from jax.experimental import pallas as pl
from jax.experimental.pallas import tpu as pltpu
from jax.experimental.pallas import tpu_sc as plsc

_IDX_CHUNK = 128


def _make_lookup(B, S, V, D):
    info = plsc.get_sparse_core_info()
    nw = info.num_cores * info.num_subcores
    seqs_per_w = B // nw
    n_gather = S // _IDX_CHUNK
    N = B * S
    lanes = info.num_lanes

    mesh = plsc.VectorSubcoreMesh(core_axis_name="c", subcore_axis_name="s")

    @functools.partial(
        pl.kernel,
        out_type=jax.ShapeDtypeStruct((N, D), jnp.float32),
        mesh=mesh,
        compiler_params=pltpu.CompilerParams(use_tc_tiling_on_sc=False),
        scratch_types=[
            pltpu.VMEM((n_gather, _IDX_CHUNK), jnp.int32),
            pltpu.VMEM((S, D), jnp.float32),
            pltpu.VMEM((S, D), jnp.float32),
            pltpu.SemaphoreType.DMA,
        ],
    )
    def lookup(x_hbm, tok_hbm, pos_hbm, out_hbm, idx_v, rows_v, pos_v, sem):
        cid = lax.axis_index("c")
        sid = lax.axis_index("s")
        wid = sid * info.num_cores + cid
        base_seq = wid * seqs_per_w

        pltpu.sync_copy(pos_hbm, pos_v)

        def seq_body(t, carry):
            sq = base_seq + t
            row0 = sq * S
            pltpu.sync_copy(x_hbm.at[pl.ds(sq * n_gather, n_gather)], idx_v)
            copies = []
            for j in range(n_gather):
                copies.append(
                    pltpu.async_copy(
                        tok_hbm.at[idx_v.at[j]],
                        rows_v.at[pl.ds(j * _IDX_CHUNK, _IDX_CHUNK)],
                        sem,
                    )
                )
            for c in copies:
                c.wait()

            def add_body(i, c):
                for h in range(D // lanes):
                    sl = pl.ds(h * lanes, lanes)
                    plsc.addupdate(rows_v.at[i, sl], pos_v[i, sl])
                return c

            lax.fori_loop(0, S, add_body, 0)
            pltpu.sync_copy(rows_v, out_hbm.at[pl.ds(row0, S)])
            return carry

        lax.fori_loop(0, seqs_per_w, seq_body, 0)

    return lookup


def kernel(x, token_table, pos_table):
    B, S = x.shape
    V, D = token_table.shape
    xf = x.reshape(B * S // _IDX_CHUNK, _IDX_CHUNK).astype(jnp.int32)
    lookup = _make_lookup(B, S, V, D)
    out = lookup(xf, token_table, pos_table)
    return out.reshape(B, S, D)


# speedup vs baseline: 1.0930x; 1.0930x over previous
import functools

import jax
import jax.numpy as jnp
from jax import lax
from jax.experimental import pallas as pl
from jax.experimental.pallas import tpu as pltpu
from jax.experimental.pallas import tpu_sc as plsc

_IDX_CHUNK = 128
_NBUF = 4
_UNROLL = 8


def _make_lookup(B, S, V, D):
    info = plsc.get_sparse_core_info()
    ncores = info.num_cores
    nw = ncores * info.num_subcores
    lanes = info.num_lanes
    seqs_per_w = B // nw
    n_gather = S // _IDX_CHUNK
    N = B * S

    mesh = plsc.VectorSubcoreMesh(core_axis_name="c", subcore_axis_name="s")

    @functools.partial(
        pl.kernel,
        out_type=jax.ShapeDtypeStruct((N, D), jnp.float32),
        mesh=mesh,
        compiler_params=pltpu.CompilerParams(use_tc_tiling_on_sc=False),
        scratch_types=[
            pltpu.VMEM((seqs_per_w * n_gather, _IDX_CHUNK), jnp.int32),
            pltpu.VMEM((_NBUF, S, D), jnp.float32),
            pltpu.VMEM((S, D), jnp.float32),
            [pltpu.SemaphoreType.DMA] * _NBUF,
        ],
    )
    def lookup(x_hbm, tok_hbm, pos_hbm, out_hbm, idx_v, rows_v, pos_v, gsems):
        cid = lax.axis_index("c")
        sid = lax.axis_index("s")
        wid = sid * ncores + cid
        base_seq = wid * seqs_per_w

        pltpu.sync_copy(
            x_hbm.at[pl.ds(base_seq * n_gather, seqs_per_w * n_gather)], idx_v
        )
        pltpu.sync_copy(pos_hbm, pos_v)

        def fire(tl, b):
            for j in range(n_gather):
                pltpu.async_copy(
                    tok_hbm.at[idx_v.at[tl * n_gather + j]],
                    rows_v.at[b, pl.ds(j * _IDX_CHUNK, _IDX_CHUNK)],
                    gsems[b],
                )

        def drain(b):
            pltpu.make_async_copy(
                tok_hbm.at[pl.ds(0, S)], rows_v.at[b], gsems[b]
            ).wait()

        def add_pos(b):
            def body(r, c):
                r0 = r * _UNROLL
                for u in range(_UNROLL):
                    for h in range(D // lanes):
                        sl = pl.ds(h * lanes, lanes)
                        plsc.addupdate(rows_v.at[b, r0 + u, sl], pos_v[r0 + u, sl])
                return c

            lax.fori_loop(0, S // _UNROLL, body, 0)

        for b in range(_NBUF):
            fire(b, b)

        def group(i, c):
            for b in range(_NBUF):
                tl = i * _NBUF + b
                drain(b)
                add_pos(b)
                pltpu.sync_copy(
                    rows_v.at[b], out_hbm.at[pl.ds((base_seq + tl) * S, S)]
                )
                fire(lax.rem(tl + _NBUF, seqs_per_w), b)
            return c

        lax.fori_loop(0, seqs_per_w // _NBUF, group, 0)
        for b in range(_NBUF):
            drain(b)

    return lookup


def kernel(x, token_table, pos_table):
    B, S = x.shape
    V, D = token_table.shape
    xf = x.reshape(B * S // _IDX_CHUNK, _IDX_CHUNK).astype(jnp.int32)
    lookup = _make_lookup(B, S, V, D)
    out = lookup(xf, token_table, pos_table)
    return out.reshape(B, S, D)
